# baseline (speedup 1.0000x reference)
import functools

import jax
import jax.numpy as jnp
from jax import lax
from jax.experimental import pallas as pl
from jax.experimental.pallas import tpu as pltpu
from jax.experimental.pallas import tpu_sc as plsc

N_NODES = 10000
D_FEAT = 128
D_HALF = 64
N_EDGES = 320000

NC = 2
NS = 16
CHUNK = 128
CHUNKS_PER_TILE = 160
E_PAD = NS * CHUNKS_PER_TILE * CHUNK
EDGE_ROWS = E_PAD // CHUNK

ACC_ROWS = 10112
XCS_ROWS = 10112
ROWS_PER_SUB = ACC_ROWS // NS
DUMP_ROW = 10000
W_ACC = 80

_SPLAT_DNUMS = lax.GatherDimensionNumbers(
    offset_dims=(), collapsed_slice_dims=(0,), start_index_map=(0,))


def _sc_scatter(xc, slab, zeros):
    mesh = plsc.VectorSubcoreMesh(core_axis_name="c", subcore_axis_name="s")

    @functools.partial(
        pl.kernel,
        out_type=jax.ShapeDtypeStruct((NC, ACC_ROWS, W_ACC), jnp.float32),
        mesh=mesh,
        scratch_types=[
            pltpu.VMEM((4, 3, CHUNK), jnp.int32),
            pltpu.VMEM((2, CHUNK, D_HALF), jnp.float32),
            pltpu.VMEM((2, CHUNK, W_ACC), jnp.float32),
            pltpu.VMEM_SHARED((XCS_ROWS, D_HALF), jnp.float32),
            pltpu.VMEM_SHARED((ACC_ROWS, W_ACC), jnp.float32),
            pltpu.SemaphoreType.DMA,
            pltpu.SemaphoreType.DMA,
            pltpu.SemaphoreType.DMA,
            pltpu.SemaphoreType.DMA,
            pltpu.SemaphoreType.DMA,
            pltpu.SemaphoreType.DMA,
            pltpu.SemaphoreType.DMA,
            pltpu.SemaphoreType.DMA,
        ],
        compiler_params=pltpu.CompilerParams(
            use_tc_tiling_on_sc=False, needs_layout_passes=False),
    )
    def k(xc_hbm, slab_hbm, z_hbm, out_hbm,
          slabv, gbuf, stg, xcs, acc,
          gs0, gs1, ss0, ss1, sl0, sl1, sl2, sl3):
        cid = lax.axis_index("c")
        sid = lax.axis_index("s")
        base = sid * CHUNKS_PER_TILE

        sl = pl.ds(sid * ROWS_PER_SUB, ROWS_PER_SUB)
        pltpu.sync_copy(z_hbm.at[sl], acc.at[sl])
        pltpu.sync_copy(xc_hbm.at[cid].at[sl], xcs.at[sl])

        ones16 = jnp.ones((16,), jnp.float32)
        for b in range(2):
            @pl.loop(0, CHUNK)
            def _(r):
                stg[b, r, pl.ds(D_HALF, 16)] = ones16

        plsc.subcore_barrier()

        gsems = (gs0, gs1)
        ssems = (ss0, ss1)
        slsems = (sl0, sl1, sl2, sl3)

        def slab_copy(j, s):
            return pltpu.make_async_copy(
                slab_hbm.at[cid].at[base + j], slabv.at[s], slsems[s])

        def gather_copy(j, s, b):
            return pltpu.make_async_copy(
                xcs.at[slabv.at[s].at[0]], gbuf.at[b], gsems[b])

        def scatter_copy(s, b):
            return pltpu.make_async_copy(
                stg.at[b], acc.at[slabv.at[s].at[1]], ssems[b])

        for s in range(4):
            slab_copy(s, s).start()
        for b in range(2):
            slab_copy(b, b).wait()
            gather_copy(b, b, b).start()

        @pl.loop(0, CHUNKS_PER_TILE, step=4)
        def _(j0):
            for b in range(4):
                j = j0 + b
                gb = b % 2

                @pl.when(j0 < CHUNKS_PER_TILE - 2 - b)
                def _():
                    slab_copy(j + 2, (b + 2) % 4).wait()

                gather_copy(j, b, gb).wait()

                if b < 2:
                    @pl.when(j0 > 0)
                    def _():
                        scatter_copy((b + 2) % 4, gb).wait()
                else:
                    scatter_copy((b + 2) % 4, gb).wait()

                @pl.loop(0, CHUNK // 16)
                def _(g):
                    wvec = plsc.bitcast(
                        slabv[b, 2, pl.ds(g * 16, 16)], jnp.float32)
                    for e16 in range(16):
                        idx = jnp.full((16, 1), e16, jnp.int32)
                        ws = lax.gather(
                            wvec, idx, _SPLAT_DNUMS, (1,),
                            mode=lax.GatherScatterMode.PROMISE_IN_BOUNDS)
                        e = g * 16 + e16
                        for kk in range(D_HALF // 16):
                            fsl = pl.ds(kk * 16, 16)
                            stg[gb, e, fsl] = gbuf[gb, e, fsl] * ws

                pltpu.async_copy(
                    stg.at[gb], acc.at[slabv.at[b].at[1]], ssems[gb],
                    add=True)

                @pl.when(j0 < CHUNKS_PER_TILE - 2 - b)
                def _():
                    gather_copy(j + 2, (b + 2) % 4, gb).start()

                @pl.when(j0 < CHUNKS_PER_TILE - 4)
                def _():
                    slab_copy(j + 4, b).start()

        for b in range(2):
            scatter_copy(b + 2, b).wait()

        plsc.subcore_barrier()
        pltpu.sync_copy(acc.at[sl], out_hbm.at[cid].at[sl])

    return k(xc, slab, zeros)


def _combine(p):
    def body(p_ref, o_ref):
        f = p_ref[0]
        b = p_ref[1]
        cf = jnp.maximum(f[:, D_HALF:D_HALF + 1], 1.0)
        cb = jnp.maximum(b[:, D_HALF:D_HALF + 1], 1.0)
        o_ref[...] = jnp.concatenate(
            [f[:, :D_HALF] / cf, b[:, :D_HALF] / cb], axis=-1)

    return pl.pallas_call(
        body,
        grid=(10,),
        in_specs=[
            pl.BlockSpec((NC, N_NODES // 10, W_ACC), lambda i: (0, i, 0)),
        ],
        out_specs=pl.BlockSpec((N_NODES // 10, D_FEAT), lambda i: (i, 0)),
        out_shape=jax.ShapeDtypeStruct((N_NODES, D_FEAT), jnp.float32),
    )(p)


def kernel(x, edge_index, edge_weight):
    x = x.astype(jnp.float32)
    row = edge_index[0].astype(jnp.int32)
    col = edge_index[1].astype(jnp.int32)
    w = edge_weight.astype(jnp.float32)

    zrows = jnp.zeros((XCS_ROWS - N_NODES, D_HALF), jnp.float32)
    xc = jnp.stack([
        jnp.concatenate([x[:, :D_HALF], zrows], axis=0),
        jnp.concatenate([x[:, D_HALF:], zrows], axis=0),
    ])
    n_pad = E_PAD - N_EDGES
    pad_i = jnp.zeros((n_pad,), jnp.int32)
    pad_d = jnp.full((n_pad,), DUMP_ROW, jnp.int32)
    pad_w = jnp.zeros((n_pad,), jnp.int32)
    wbits = lax.bitcast_convert_type(w, jnp.int32)
    src2 = jnp.stack([
        jnp.concatenate([col, pad_i]),
        jnp.concatenate([row, pad_i]),
    ]).reshape(NC, EDGE_ROWS, CHUNK)
    dst2 = jnp.stack([
        jnp.concatenate([row, pad_d]),
        jnp.concatenate([col, pad_d]),
    ]).reshape(NC, EDGE_ROWS, CHUNK)
    w2 = jnp.stack([
        jnp.concatenate([wbits, pad_w]),
        jnp.concatenate([wbits, pad_w]),
    ]).reshape(NC, EDGE_ROWS, CHUNK)
    slab = jnp.stack([src2, dst2, w2], axis=2)
    zeros = jnp.zeros((ACC_ROWS, W_ACC), jnp.float32)

    partials = _sc_scatter(xc, slab, zeros)
    return _combine(partials[:, :N_NODES, :])


# speedup vs baseline: 1.2195x; 1.2195x over previous
import functools

import jax
import jax.numpy as jnp
from jax import lax
from jax.experimental import pallas as pl
from jax.experimental.pallas import tpu as pltpu
from jax.experimental.pallas import tpu_sc as plsc

N_NODES = 10000
D_FEAT = 128
D_HALF = 64
N_EDGES = 320000

NC = 2
NS = 16
CHUNK = 128
CHUNKS_PER_TILE = 160
E_PAD = NS * CHUNKS_PER_TILE * CHUNK
EDGE_ROWS = E_PAD // CHUNK

ACC_ROWS = 10112
XCS_ROWS = 10112
ROWS_PER_SUB = ACC_ROWS // NS
DUMP_ROW = 10000
W_ACC = 80

_SPLAT_DNUMS = lax.GatherDimensionNumbers(
    offset_dims=(), collapsed_slice_dims=(0,), start_index_map=(0,))


def _sc_scatter(xc, slab, zeros):
    mesh = plsc.VectorSubcoreMesh(core_axis_name="c", subcore_axis_name="s")

    @functools.partial(
        pl.kernel,
        out_type=jax.ShapeDtypeStruct((NC, ACC_ROWS, W_ACC), jnp.float32),
        mesh=mesh,
        scratch_types=[
            pltpu.VMEM((4, 3, CHUNK), jnp.int32),
            pltpu.VMEM((2, CHUNK, D_HALF), jnp.float32),
            pltpu.VMEM((2, CHUNK, W_ACC), jnp.float32),
            pltpu.VMEM_SHARED((XCS_ROWS, D_HALF), jnp.float32),
            pltpu.VMEM_SHARED((ACC_ROWS, W_ACC), jnp.float32),
            pltpu.SemaphoreType.DMA,
            pltpu.SemaphoreType.DMA,
            pltpu.SemaphoreType.DMA,
            pltpu.SemaphoreType.DMA,
            pltpu.SemaphoreType.DMA,
            pltpu.SemaphoreType.DMA,
            pltpu.SemaphoreType.DMA,
            pltpu.SemaphoreType.DMA,
        ],
        compiler_params=pltpu.CompilerParams(
            use_tc_tiling_on_sc=False, needs_layout_passes=False),
    )
    def k(xc_hbm, slab_hbm, z_hbm, out_hbm,
          slabv, gbuf, stg, xcs, acc,
          gs0, gs1, ss0, ss1, sl0, sl1, sl2, sl3):
        cid = lax.axis_index("c")
        sid = lax.axis_index("s")
        base = sid * CHUNKS_PER_TILE

        sl = pl.ds(sid * ROWS_PER_SUB, ROWS_PER_SUB)
        pltpu.sync_copy(z_hbm.at[sl], acc.at[sl])
        pltpu.sync_copy(xc_hbm.at[cid].at[sl], xcs.at[sl])

        ones16 = jnp.ones((16,), jnp.float32)
        for b in range(2):
            @pl.loop(0, CHUNK)
            def _(r):
                stg[b, r, pl.ds(D_HALF, 16)] = ones16

        plsc.subcore_barrier()

        gsems = (gs0, gs1)
        ssems = (ss0, ss1)
        slsems = (sl0, sl1, sl2, sl3)

        def slab_copy(j, s):
            return pltpu.make_async_copy(
                slab_hbm.at[cid].at[base + j], slabv.at[s], slsems[s])

        def gather_copy(j, s, b):
            return pltpu.make_async_copy(
                xcs.at[slabv.at[s].at[0]], gbuf.at[b], gsems[b])

        def scatter_copy(s, b):
            return pltpu.make_async_copy(
                stg.at[b], acc.at[slabv.at[s].at[1]], ssems[b])

        for s in range(4):
            slab_copy(s, s).start()
        for b in range(2):
            slab_copy(b, b).wait()
            gather_copy(b, b, b).start()

        @pl.loop(0, CHUNKS_PER_TILE, step=4)
        def _(j0):
            for b in range(4):
                j = j0 + b
                gb = b % 2

                @pl.when(j0 < CHUNKS_PER_TILE - 2 - b)
                def _():
                    slab_copy(j + 2, (b + 2) % 4).wait()

                gather_copy(j, b, gb).wait()

                if b < 2:
                    @pl.when(j0 > 0)
                    def _():
                        scatter_copy((b + 2) % 4, gb).wait()
                else:
                    scatter_copy((b + 2) % 4, gb).wait()

                @plsc.parallel_loop(0, CHUNK // 16, unroll=2)
                def _(g):
                    wvec = plsc.bitcast(
                        slabv[b, 2, pl.ds(g * 16, 16)], jnp.float32)
                    for e16 in range(16):
                        idx = jnp.full((16, 1), e16, jnp.int32)
                        ws = lax.gather(
                            wvec, idx, _SPLAT_DNUMS, (1,),
                            mode=lax.GatherScatterMode.PROMISE_IN_BOUNDS)
                        e = g * 16 + e16
                        for kk in range(D_HALF // 16):
                            fsl = pl.ds(kk * 16, 16)
                            stg[gb, e, fsl] = gbuf[gb, e, fsl] * ws

                pltpu.async_copy(
                    stg.at[gb], acc.at[slabv.at[b].at[1]], ssems[gb],
                    add=True)

                @pl.when(j0 < CHUNKS_PER_TILE - 2 - b)
                def _():
                    gather_copy(j + 2, (b + 2) % 4, gb).start()

                @pl.when(j0 < CHUNKS_PER_TILE - 4)
                def _():
                    slab_copy(j + 4, b).start()

        for b in range(2):
            scatter_copy(b + 2, b).wait()

        plsc.subcore_barrier()
        pltpu.sync_copy(acc.at[sl], out_hbm.at[cid].at[sl])

    return k(xc, slab, zeros)


def _combine(p):
    def body(p_ref, o_ref):
        f = p_ref[0]
        b = p_ref[1]
        cf = jnp.maximum(f[:, D_HALF:D_HALF + 1], 1.0)
        cb = jnp.maximum(b[:, D_HALF:D_HALF + 1], 1.0)
        o_ref[...] = jnp.concatenate(
            [f[:, :D_HALF] / cf, b[:, :D_HALF] / cb], axis=-1)

    return pl.pallas_call(
        body,
        grid=(10,),
        in_specs=[
            pl.BlockSpec((NC, N_NODES // 10, W_ACC), lambda i: (0, i, 0)),
        ],
        out_specs=pl.BlockSpec((N_NODES // 10, D_FEAT), lambda i: (i, 0)),
        out_shape=jax.ShapeDtypeStruct((N_NODES, D_FEAT), jnp.float32),
    )(p)


def kernel(x, edge_index, edge_weight):
    x = x.astype(jnp.float32)
    row = edge_index[0].astype(jnp.int32)
    col = edge_index[1].astype(jnp.int32)
    w = edge_weight.astype(jnp.float32)

    zrows = jnp.zeros((XCS_ROWS - N_NODES, D_HALF), jnp.float32)
    xc = jnp.stack([
        jnp.concatenate([x[:, :D_HALF], zrows], axis=0),
        jnp.concatenate([x[:, D_HALF:], zrows], axis=0),
    ])
    n_pad = E_PAD - N_EDGES
    pad_i = jnp.zeros((n_pad,), jnp.int32)
    pad_d = jnp.full((n_pad,), DUMP_ROW, jnp.int32)
    pad_w = jnp.zeros((n_pad,), jnp.int32)
    wbits = lax.bitcast_convert_type(w, jnp.int32)
    src2 = jnp.stack([
        jnp.concatenate([col, pad_i]),
        jnp.concatenate([row, pad_i]),
    ]).reshape(NC, EDGE_ROWS, CHUNK)
    dst2 = jnp.stack([
        jnp.concatenate([row, pad_d]),
        jnp.concatenate([col, pad_d]),
    ]).reshape(NC, EDGE_ROWS, CHUNK)
    w2 = jnp.stack([
        jnp.concatenate([wbits, pad_w]),
        jnp.concatenate([wbits, pad_w]),
    ]).reshape(NC, EDGE_ROWS, CHUNK)
    slab = jnp.stack([src2, dst2, w2], axis=2)
    zeros = jnp.zeros((ACC_ROWS, W_ACC), jnp.float32)

    partials = _sc_scatter(xc, slab, zeros)
    return _combine(partials[:, :N_NODES, :])


# speedup vs baseline: 1.5970x; 1.3095x over previous
import functools

import jax
import jax.numpy as jnp
from jax import lax
from jax.experimental import pallas as pl
from jax.experimental.pallas import tpu as pltpu
from jax.experimental.pallas import tpu_sc as plsc

N_NODES = 10000
D_FEAT = 128
D_HALF = 64
N_EDGES = 320000

NC = 2
NS = 16
CHUNK = 128
CHUNKS_PER_TILE = 160
E_PAD = NS * CHUNKS_PER_TILE * CHUNK
EDGE_ROWS = E_PAD // CHUNK

ACC_ROWS = 10112
XCS_ROWS = 10112
ROWS_PER_SUB = ACC_ROWS // NS
DUMP_ROW = 10000
W_ACC = 80

_SPLAT_DNUMS = lax.GatherDimensionNumbers(
    offset_dims=(), collapsed_slice_dims=(0,), start_index_map=(0,))


def _sc_scatter(xc, slab, zeros):
    mesh = plsc.VectorSubcoreMesh(core_axis_name="c", subcore_axis_name="s")

    @functools.partial(
        pl.kernel,
        out_type=jax.ShapeDtypeStruct((NC, ACC_ROWS, W_ACC), jnp.float32),
        mesh=mesh,
        scratch_types=[
            pltpu.VMEM((4, 3, CHUNK), jnp.int32),
            pltpu.VMEM((2, CHUNK, D_HALF), jnp.float32),
            pltpu.VMEM((2, CHUNK, W_ACC), jnp.float32),
            pltpu.VMEM_SHARED((XCS_ROWS, D_HALF), jnp.float32),
            pltpu.VMEM_SHARED((ACC_ROWS, W_ACC), jnp.float32),
            pltpu.SemaphoreType.DMA,
            pltpu.SemaphoreType.DMA,
            pltpu.SemaphoreType.DMA,
            pltpu.SemaphoreType.DMA,
            pltpu.SemaphoreType.DMA,
            pltpu.SemaphoreType.DMA,
            pltpu.SemaphoreType.DMA,
            pltpu.SemaphoreType.DMA,
        ],
        compiler_params=pltpu.CompilerParams(
            use_tc_tiling_on_sc=False, needs_layout_passes=False),
    )
    def k(xc_hbm, slab_hbm, z_hbm, out_hbm,
          slabv, gbuf, stg, xcs, acc,
          gs0, gs1, ss0, ss1, sl0, sl1, sl2, sl3):
        cid = lax.axis_index("c")
        sid = lax.axis_index("s")
        base = sid * CHUNKS_PER_TILE

        sl = pl.ds(sid * ROWS_PER_SUB, ROWS_PER_SUB)
        pltpu.sync_copy(z_hbm.at[sl], acc.at[sl])
        pltpu.sync_copy(xc_hbm.at[cid].at[sl], xcs.at[sl])

        ones16 = jnp.ones((16,), jnp.float32)
        for b in range(2):
            @pl.loop(0, CHUNK)
            def _(r):
                stg[b, r, pl.ds(D_HALF, 16)] = ones16

        plsc.subcore_barrier()

        gsems = (gs0, gs1)
        ssems = (ss0, ss1)
        slsems = (sl0, sl1, sl2, sl3)

        def slab_copy(j, s):
            return pltpu.make_async_copy(
                slab_hbm.at[cid].at[base + j], slabv.at[s], slsems[s])

        def gather_copy(j, s, b):
            return pltpu.make_async_copy(
                xcs.at[slabv.at[s].at[0]], gbuf.at[b], gsems[b])

        def scatter_copy(s, b):
            return pltpu.make_async_copy(
                stg.at[b], acc.at[slabv.at[s].at[1]], ssems[b])

        for s in range(4):
            slab_copy(s, s).start()
        for b in range(2):
            slab_copy(b, b).wait()
            gather_copy(b, b, b).start()

        @pl.loop(0, CHUNKS_PER_TILE, step=4)
        def _(j0):
            for b in range(4):
                j = j0 + b
                gb = b % 2

                @pl.when(j0 < CHUNKS_PER_TILE - 2 - b)
                def _():
                    slab_copy(j + 2, (b + 2) % 4).wait()

                gather_copy(j, b, gb).wait()

                if b < 2:
                    @pl.when(j0 > 0)
                    def _():
                        scatter_copy((b + 2) % 4, gb).wait()
                else:
                    scatter_copy((b + 2) % 4, gb).wait()

                @plsc.parallel_loop(0, CHUNK // 16, unroll=4)
                def _(g):
                    wvec = plsc.bitcast(
                        slabv[b, 2, pl.ds(g * 16, 16)], jnp.float32)
                    for e16 in range(16):
                        idx = jnp.full((16, 1), e16, jnp.int32)
                        ws = lax.gather(
                            wvec, idx, _SPLAT_DNUMS, (1,),
                            mode=lax.GatherScatterMode.PROMISE_IN_BOUNDS)
                        e = g * 16 + e16
                        for kk in range(D_HALF // 16):
                            fsl = pl.ds(kk * 16, 16)
                            stg[gb, e, fsl] = gbuf[gb, e, fsl] * ws

                pltpu.async_copy(
                    stg.at[gb], acc.at[slabv.at[b].at[1]], ssems[gb],
                    add=True)

                @pl.when(j0 < CHUNKS_PER_TILE - 2 - b)
                def _():
                    gather_copy(j + 2, (b + 2) % 4, gb).start()

                @pl.when(j0 < CHUNKS_PER_TILE - 4)
                def _():
                    slab_copy(j + 4, b).start()

        for b in range(2):
            scatter_copy(b + 2, b).wait()

        plsc.subcore_barrier()
        pltpu.sync_copy(acc.at[sl], out_hbm.at[cid].at[sl])

    return k(xc, slab, zeros)


def _combine(p):
    def body(p_ref, o_ref):
        f = p_ref[0]
        b = p_ref[1]
        cf = jnp.maximum(f[:, D_HALF:D_HALF + 1], 1.0)
        cb = jnp.maximum(b[:, D_HALF:D_HALF + 1], 1.0)
        o_ref[...] = jnp.concatenate(
            [f[:, :D_HALF] / cf, b[:, :D_HALF] / cb], axis=-1)

    return pl.pallas_call(
        body,
        grid=(10,),
        in_specs=[
            pl.BlockSpec((NC, N_NODES // 10, W_ACC), lambda i: (0, i, 0)),
        ],
        out_specs=pl.BlockSpec((N_NODES // 10, D_FEAT), lambda i: (i, 0)),
        out_shape=jax.ShapeDtypeStruct((N_NODES, D_FEAT), jnp.float32),
    )(p)


def kernel(x, edge_index, edge_weight):
    x = x.astype(jnp.float32)
    row = edge_index[0].astype(jnp.int32)
    col = edge_index[1].astype(jnp.int32)
    w = edge_weight.astype(jnp.float32)

    zrows = jnp.zeros((XCS_ROWS - N_NODES, D_HALF), jnp.float32)
    xc = jnp.stack([
        jnp.concatenate([x[:, :D_HALF], zrows], axis=0),
        jnp.concatenate([x[:, D_HALF:], zrows], axis=0),
    ])
    n_pad = E_PAD - N_EDGES
    pad_i = jnp.zeros((n_pad,), jnp.int32)
    pad_d = jnp.full((n_pad,), DUMP_ROW, jnp.int32)
    pad_w = jnp.zeros((n_pad,), jnp.int32)
    wbits = lax.bitcast_convert_type(w, jnp.int32)
    src2 = jnp.stack([
        jnp.concatenate([col, pad_i]),
        jnp.concatenate([row, pad_i]),
    ]).reshape(NC, EDGE_ROWS, CHUNK)
    dst2 = jnp.stack([
        jnp.concatenate([row, pad_d]),
        jnp.concatenate([col, pad_d]),
    ]).reshape(NC, EDGE_ROWS, CHUNK)
    w2 = jnp.stack([
        jnp.concatenate([wbits, pad_w]),
        jnp.concatenate([wbits, pad_w]),
    ]).reshape(NC, EDGE_ROWS, CHUNK)
    slab = jnp.stack([src2, dst2, w2], axis=2)
    zeros = jnp.zeros((ACC_ROWS, W_ACC), jnp.float32)

    partials = _sc_scatter(xc, slab, zeros)
    return _combine(partials[:, :N_NODES, :])


# speedup vs baseline: 1.6146x; 1.0110x over previous
import functools

import jax
import jax.numpy as jnp
from jax import lax
from jax.experimental import pallas as pl
from jax.experimental.pallas import tpu as pltpu
from jax.experimental.pallas import tpu_sc as plsc

N_NODES = 10000
D_FEAT = 128
D_HALF = 64
N_EDGES = 320000

NC = 2
NS = 16
CHUNK = 128
CHUNKS_PER_TILE = 160
E_PAD = NS * CHUNKS_PER_TILE * CHUNK
EDGE_ROWS = E_PAD // CHUNK

ACC_ROWS = 10112
XCS_ROWS = 10112
ROWS_PER_SUB = ACC_ROWS // NS
DUMP_ROW = 10000
W_ACC = 80

_SPLAT_DNUMS = lax.GatherDimensionNumbers(
    offset_dims=(), collapsed_slice_dims=(0,), start_index_map=(0,))


def _sc_scatter(xc, slab, zeros):
    mesh = plsc.VectorSubcoreMesh(core_axis_name="c", subcore_axis_name="s")

    @functools.partial(
        pl.kernel,
        out_type=jax.ShapeDtypeStruct((NC, ACC_ROWS, W_ACC), jnp.float32),
        mesh=mesh,
        scratch_types=[
            pltpu.VMEM((4, 3, CHUNK), jnp.int32),
            pltpu.VMEM((2, CHUNK, D_HALF), jnp.float32),
            pltpu.VMEM((2, CHUNK, W_ACC), jnp.float32),
            pltpu.VMEM_SHARED((XCS_ROWS, D_HALF), jnp.float32),
            pltpu.VMEM_SHARED((ACC_ROWS, W_ACC), jnp.float32),
            pltpu.SemaphoreType.DMA,
            pltpu.SemaphoreType.DMA,
            pltpu.SemaphoreType.DMA,
            pltpu.SemaphoreType.DMA,
            pltpu.SemaphoreType.DMA,
            pltpu.SemaphoreType.DMA,
            pltpu.SemaphoreType.DMA,
            pltpu.SemaphoreType.DMA,
        ],
        compiler_params=pltpu.CompilerParams(
            use_tc_tiling_on_sc=False, needs_layout_passes=False),
    )
    def k(xc_hbm, slab_hbm, z_hbm, out_hbm,
          slabv, gbuf, stg, xcs, acc,
          gs0, gs1, ss0, ss1, sl0, sl1, sl2, sl3):
        cid = lax.axis_index("c")
        sid = lax.axis_index("s")
        base = sid * CHUNKS_PER_TILE

        sl = pl.ds(sid * ROWS_PER_SUB, ROWS_PER_SUB)
        pltpu.sync_copy(z_hbm.at[sl], acc.at[sl])
        pltpu.sync_copy(xc_hbm.at[cid].at[sl], xcs.at[sl])

        ones16 = jnp.ones((16,), jnp.float32)
        for b in range(2):
            @pl.loop(0, CHUNK)
            def _(r):
                stg[b, r, pl.ds(D_HALF, 16)] = ones16

        plsc.subcore_barrier()

        gsems = (gs0, gs1)
        ssems = (ss0, ss1)
        slsems = (sl0, sl1, sl2, sl3)

        def slab_copy(j, s):
            return pltpu.make_async_copy(
                slab_hbm.at[cid].at[base + j], slabv.at[s], slsems[s])

        def gather_copy(j, s, b):
            return pltpu.make_async_copy(
                xcs.at[slabv.at[s].at[0]], gbuf.at[b], gsems[b])

        def scatter_copy(s, b):
            return pltpu.make_async_copy(
                stg.at[b], acc.at[slabv.at[s].at[1]], ssems[b])

        for s in range(4):
            slab_copy(s, s).start()
        for b in range(2):
            slab_copy(b, b).wait()
            gather_copy(b, b, b).start()

        @pl.loop(0, CHUNKS_PER_TILE, step=4)
        def _(j0):
            for b in range(4):
                j = j0 + b
                gb = b % 2

                @pl.when(j0 < CHUNKS_PER_TILE - 2 - b)
                def _():
                    slab_copy(j + 2, (b + 2) % 4).wait()

                gather_copy(j, b, gb).wait()

                if b < 2:
                    @pl.when(j0 > 0)
                    def _():
                        scatter_copy((b + 2) % 4, gb).wait()
                else:
                    scatter_copy((b + 2) % 4, gb).wait()

                @plsc.parallel_loop(0, CHUNK // 16, unroll=8)
                def _(g):
                    wvec = plsc.bitcast(
                        slabv[b, 2, pl.ds(g * 16, 16)], jnp.float32)
                    for e16 in range(16):
                        idx = jnp.full((16, 1), e16, jnp.int32)
                        ws = lax.gather(
                            wvec, idx, _SPLAT_DNUMS, (1,),
                            mode=lax.GatherScatterMode.PROMISE_IN_BOUNDS)
                        e = g * 16 + e16
                        for kk in range(D_HALF // 16):
                            fsl = pl.ds(kk * 16, 16)
                            stg[gb, e, fsl] = gbuf[gb, e, fsl] * ws

                pltpu.async_copy(
                    stg.at[gb], acc.at[slabv.at[b].at[1]], ssems[gb],
                    add=True)

                @pl.when(j0 < CHUNKS_PER_TILE - 2 - b)
                def _():
                    gather_copy(j + 2, (b + 2) % 4, gb).start()

                @pl.when(j0 < CHUNKS_PER_TILE - 4)
                def _():
                    slab_copy(j + 4, b).start()

        for b in range(2):
            scatter_copy(b + 2, b).wait()

        plsc.subcore_barrier()
        pltpu.sync_copy(acc.at[sl], out_hbm.at[cid].at[sl])

    return k(xc, slab, zeros)


def _combine(p):
    def body(p_ref, o_ref):
        f = p_ref[0]
        b = p_ref[1]
        cf = jnp.maximum(f[:, D_HALF:D_HALF + 1], 1.0)
        cb = jnp.maximum(b[:, D_HALF:D_HALF + 1], 1.0)
        o_ref[...] = jnp.concatenate(
            [f[:, :D_HALF] / cf, b[:, :D_HALF] / cb], axis=-1)

    return pl.pallas_call(
        body,
        grid=(10,),
        in_specs=[
            pl.BlockSpec((NC, N_NODES // 10, W_ACC), lambda i: (0, i, 0)),
        ],
        out_specs=pl.BlockSpec((N_NODES // 10, D_FEAT), lambda i: (i, 0)),
        out_shape=jax.ShapeDtypeStruct((N_NODES, D_FEAT), jnp.float32),
    )(p)


def kernel(x, edge_index, edge_weight):
    x = x.astype(jnp.float32)
    row = edge_index[0].astype(jnp.int32)
    col = edge_index[1].astype(jnp.int32)
    w = edge_weight.astype(jnp.float32)

    zrows = jnp.zeros((XCS_ROWS - N_NODES, D_HALF), jnp.float32)
    xc = jnp.stack([
        jnp.concatenate([x[:, :D_HALF], zrows], axis=0),
        jnp.concatenate([x[:, D_HALF:], zrows], axis=0),
    ])
    n_pad = E_PAD - N_EDGES
    pad_i = jnp.zeros((n_pad,), jnp.int32)
    pad_d = jnp.full((n_pad,), DUMP_ROW, jnp.int32)
    pad_w = jnp.zeros((n_pad,), jnp.int32)
    wbits = lax.bitcast_convert_type(w, jnp.int32)
    src2 = jnp.stack([
        jnp.concatenate([col, pad_i]),
        jnp.concatenate([row, pad_i]),
    ]).reshape(NC, EDGE_ROWS, CHUNK)
    dst2 = jnp.stack([
        jnp.concatenate([row, pad_d]),
        jnp.concatenate([col, pad_d]),
    ]).reshape(NC, EDGE_ROWS, CHUNK)
    w2 = jnp.stack([
        jnp.concatenate([wbits, pad_w]),
        jnp.concatenate([wbits, pad_w]),
    ]).reshape(NC, EDGE_ROWS, CHUNK)
    slab = jnp.stack([src2, dst2, w2], axis=2)
    zeros = jnp.zeros((ACC_ROWS, W_ACC), jnp.float32)

    partials = _sc_scatter(xc, slab, zeros)
    return _combine(partials[:, :N_NODES, :])
